# baseline (speedup 1.0000x reference)
import functools

import jax
import jax.numpy as jnp
from jax import lax
from jax.experimental import pallas as pl
from jax.experimental.pallas import tpu as pltpu
from jax.experimental.pallas import tpu_sc as plsc

B = 4096
NF = 26
V = 100001
D = 32
BNF = B * NF
NC, NS = 2, 16
NW = NC * NS
BPW = BNF // NW
CHUNK = 128
NCH = BPW // CHUNK
DG = 16
NDR = (NF * V + DG - 1) // DG + 1

BLK = 512
HID0, HID1, HID2 = 1024, 512, 256


@functools.cache
def _build_sc_gather():
    mesh = plsc.VectorSubcoreMesh(core_axis_name="c", subcore_axis_name="s")

    @functools.partial(
        pl.kernel,
        out_type=(
            jax.ShapeDtypeStruct((BNF, D), jnp.bfloat16),
            jax.ShapeDtypeStruct((BNF, DG), jnp.float32),
        ),
        mesh=mesh,
        scratch_types=[
            pltpu.VMEM((NCH, CHUNK), jnp.int32),
            pltpu.VMEM((NCH, CHUNK), jnp.int32),
            pltpu.VMEM((BPW, D), jnp.bfloat16),
            pltpu.VMEM((CHUNK, DG), jnp.float32),
            pltpu.SemaphoreType.DMA,
            pltpu.SemaphoreType.DMA,
        ],
        compiler_params=pltpu.CompilerParams(use_tc_tiling_on_sc=False),
    )
    def _sc_gather(emb_hbm, dense_hbm, idx_hbm, idxd_hbm, oe_hbm, od_hbm,
                   idx_v, idxd_v, rows_v, db_v, sem_e, sem_d):
        wid = lax.axis_index("s") * NC + lax.axis_index("c")
        base = wid * BPW
        pltpu.sync_copy(idx_hbm.at[wid], idx_v)
        pltpu.sync_copy(idxd_hbm.at[wid], idxd_v)

        @pl.loop(0, NCH)
        def _(j):
            off = j * CHUNK
            cpe = pltpu.async_copy(
                emb_hbm.at[idx_v.at[j]], rows_v.at[pl.ds(off, CHUNK)], sem_e)
            cpd = pltpu.async_copy(dense_hbm.at[idxd_v.at[j]], db_v, sem_d)
            cpe.wait()
            cpd.wait()
            pltpu.sync_copy(db_v, od_hbm.at[pl.ds(base + off, CHUNK)])

        pltpu.sync_copy(rows_v, oe_hbm.at[pl.ds(base, BPW)])

    return _sc_gather


def _mlp_body(h0_ref, dv_ref, xm_ref, fmb_ref, w0_ref, b0_ref, w1_ref, b1_ref,
              w2_ref, b2_ref, w3_ref, b3_ref, out_ref):
    h0 = h0_ref[...]
    h0f = h0.astype(jnp.float32)
    fi = lax.broadcasted_iota(jnp.int32, (NF, NF * DG), 0)
    fj = lax.broadcasted_iota(jnp.int32, (NF, NF * DG), 1)
    expand = ((fj // DG) == fi).astype(jnp.float32)
    xm = jnp.dot(xm_ref[...].astype(jnp.float32), expand,
                 preferred_element_type=jnp.float32)
    lane = lax.broadcasted_iota(jnp.int32, (BLK, NF * DG), 1) % DG
    onehot = (lane == xm.astype(jnp.int32)).astype(jnp.float32)
    one = jnp.sum(dv_ref[...] * onehot, axis=1)
    ssq = jnp.sum(h0f * h0f, axis=1)
    ki = lax.broadcasted_iota(jnp.int32, (NF * D, D), 0)
    di = lax.broadcasted_iota(jnp.int32, (NF * D, D), 1)
    fold = (ki % D == di).astype(jnp.bfloat16)
    s = jnp.dot(h0, fold, preferred_element_type=jnp.float32)
    two = 0.5 * (jnp.sum(s * s, axis=1) - ssq)
    fm = one + two + fmb_ref[...]

    h = h0
    for w_ref, b_ref in ((w0_ref, b0_ref), (w1_ref, b1_ref), (w2_ref, b2_ref)):
        hf = jnp.dot(h, w_ref[...], preferred_element_type=jnp.float32)
        h = jnp.maximum(hf + b_ref[...][None, :], 0.0).astype(jnp.bfloat16)
    hf = h.astype(jnp.float32)
    deep = jnp.maximum(jnp.sum(hf * w3_ref[...], axis=1) + b3_ref[...], 0.0)
    out_ref[...] = jax.nn.sigmoid(fm + deep)


_mlp = pl.pallas_call(
    _mlp_body,
    grid=(B // BLK,),
    in_specs=[
        pl.BlockSpec((BLK, NF * D), lambda i: (i, 0)),
        pl.BlockSpec((BLK, NF * DG), lambda i: (i, 0)),
        pl.BlockSpec((BLK, NF), lambda i: (i, 0)),
        pl.BlockSpec((1,), lambda i: (0,)),
        pl.BlockSpec((NF * D, HID0), lambda i: (0, 0)),
        pl.BlockSpec((HID0,), lambda i: (0,)),
        pl.BlockSpec((HID0, HID1), lambda i: (0, 0)),
        pl.BlockSpec((HID1,), lambda i: (0,)),
        pl.BlockSpec((HID1, HID2), lambda i: (0, 0)),
        pl.BlockSpec((HID2,), lambda i: (0,)),
        pl.BlockSpec((1, HID2), lambda i: (0, 0)),
        pl.BlockSpec((1,), lambda i: (0,)),
    ],
    out_specs=pl.BlockSpec((BLK,), lambda i: (i,)),
    out_shape=jax.ShapeDtypeStruct((B,), jnp.float32),
)


def kernel(X, emb_table, dense_table, fm_bias, W0, b0, W1, b1, W2, b2, W3, b3):
    X = X.astype(jnp.int32)
    idx = X + (jnp.arange(NF, dtype=jnp.int32) * V)[None, :]
    idx3 = idx.reshape(NW, NCH, CHUNK)
    idxd3 = (idx // DG).reshape(NW, NCH, CHUNK)
    xmod = idx % DG
    emb2d = emb_table.reshape(NF * V, D).astype(jnp.bfloat16)
    dense_lin = jnp.pad(dense_table.reshape(NF * V), (0, NDR * DG - NF * V))
    dense2d = dense_lin.reshape(NDR, DG)
    rows, dvals = _build_sc_gather()(emb2d, dense2d, idx3, idxd3)
    h0 = rows.reshape(B, NF * D)
    dv = dvals.reshape(B, NF * DG)
    return _mlp(h0, dv, xmod, fm_bias,
                W0.astype(jnp.bfloat16), b0, W1.astype(jnp.bfloat16), b1,
                W2.astype(jnp.bfloat16), b2, W3.reshape(1, HID2), b3)


# speedup vs baseline: 1.5381x; 1.5381x over previous
import functools

import jax
import jax.numpy as jnp
from jax import lax
from jax.experimental import pallas as pl
from jax.experimental.pallas import tpu as pltpu
from jax.experimental.pallas import tpu_sc as plsc

B = 4096
NF = 26
V = 100001
D = 32
BNF = B * NF
NC, NS = 2, 16
NW = NC * NS
BPW = BNF // NW
CHUNK = 128
NCH = BPW // CHUNK

XCH = 512
NXC = (V + XCH - 1) // XCH
QPF = NXC * CHUNK
EPR = NF * QPF
NDR = (NF * V + CHUNK - 1) // CHUNK + 1

BLK = 512
HID0, HID1, HID2 = 1024, 512, 256
LW = NF * CHUNK


def _repack_body(i_ref, o_ref):
    x = i_ref[...]
    x3 = x.reshape(D, XCH // 4, 4)
    o_ref[...] = jnp.transpose(x3, (1, 2, 0)).reshape(CHUNK, CHUNK)


_repack = pl.pallas_call(
    _repack_body,
    grid=(NF, NXC),
    in_specs=[pl.BlockSpec((D, XCH), lambda f, c: (f, c))],
    out_specs=pl.BlockSpec((CHUNK, CHUNK), lambda f, c: (f * NXC + c, 0)),
    out_shape=jax.ShapeDtypeStruct((EPR, CHUNK), jnp.float32),
)


@functools.cache
def _build_sc_gather():
    mesh = plsc.VectorSubcoreMesh(core_axis_name="c", subcore_axis_name="s")

    @functools.partial(
        pl.kernel,
        out_type=(
            jax.ShapeDtypeStruct((BNF, CHUNK), jnp.float32),
            jax.ShapeDtypeStruct((BNF, CHUNK), jnp.float32),
        ),
        mesh=mesh,
        scratch_types=[
            pltpu.VMEM((NCH, CHUNK), jnp.int32),
            pltpu.VMEM((NCH, CHUNK), jnp.int32),
            pltpu.VMEM((CHUNK, CHUNK), jnp.float32),
            pltpu.VMEM((CHUNK, CHUNK), jnp.float32),
            pltpu.SemaphoreType.DMA,
            pltpu.SemaphoreType.DMA,
        ],
    )
    def _sc_gather(emb_hbm, dense_hbm, idx_hbm, idxd_hbm, oe_hbm, od_hbm,
                   idx_v, idxd_v, qb_v, db_v, sem_e, sem_d):
        wid = lax.axis_index("s") * NC + lax.axis_index("c")
        base = wid * BPW
        pltpu.sync_copy(idx_hbm.at[wid], idx_v)
        pltpu.sync_copy(idxd_hbm.at[wid], idxd_v)

        @pl.loop(0, NCH)
        def _(j):
            off = base + j * CHUNK
            cpe = pltpu.async_copy(emb_hbm.at[idx_v.at[j]], qb_v, sem_e)
            cpd = pltpu.async_copy(dense_hbm.at[idxd_v.at[j]], db_v, sem_d)
            cpe.wait()
            cpd.wait()
            pltpu.sync_copy(qb_v, oe_hbm.at[pl.ds(off, CHUNK)])
            pltpu.sync_copy(db_v, od_hbm.at[pl.ds(off, CHUNK)])

    return _sc_gather


def _mlp_body(hq_ref, dq_ref, xa_ref, xm_ref, exf_ref, m2_ref, fmb_ref,
              w0_ref, b0_ref, w1_ref, b1_ref, w2_ref, b2_ref, w3_ref, b3_ref,
              out_ref):
    exf = exf_ref[...]
    lane = lax.broadcasted_iota(jnp.int32, (BLK, LW), 1) % CHUNK
    a_exp = jnp.dot(xa_ref[...].astype(jnp.float32), exf,
                    preferred_element_type=jnp.float32)
    emask = (lane // D == a_exp.astype(jnp.int32)).astype(jnp.float32)
    hsel = (hq_ref[...] * emask).astype(jnp.bfloat16)
    h0 = jnp.dot(hsel, m2_ref[...], preferred_element_type=jnp.float32)
    m_exp = jnp.dot(xm_ref[...].astype(jnp.float32), exf,
                    preferred_element_type=jnp.float32)
    dmask = (lane == m_exp.astype(jnp.int32)).astype(jnp.float32)
    one = jnp.sum(dq_ref[...] * dmask, axis=1)
    ssq = jnp.sum(h0 * h0, axis=1)
    ki = lax.broadcasted_iota(jnp.int32, (NF * D, D), 0)
    di = lax.broadcasted_iota(jnp.int32, (NF * D, D), 1)
    fold = (ki % D == di).astype(jnp.bfloat16)
    h0b = h0.astype(jnp.bfloat16)
    s = jnp.dot(h0b, fold, preferred_element_type=jnp.float32)
    two = 0.5 * (jnp.sum(s * s, axis=1) - ssq)
    fm = one + two + fmb_ref[...]
    h = h0b
    for w_ref, b_ref in ((w0_ref, b0_ref), (w1_ref, b1_ref), (w2_ref, b2_ref)):
        hf = jnp.dot(h, w_ref[...], preferred_element_type=jnp.float32)
        h = jnp.maximum(hf + b_ref[...][None, :], 0.0).astype(jnp.bfloat16)
    hf = h.astype(jnp.float32)
    deep = jnp.maximum(jnp.sum(hf * w3_ref[...], axis=1) + b3_ref[...], 0.0)
    out_ref[...] = jax.nn.sigmoid(fm + deep)


_mlp = pl.pallas_call(
    _mlp_body,
    grid=(B // BLK,),
    in_specs=[
        pl.BlockSpec((BLK, LW), lambda i: (i, 0)),
        pl.BlockSpec((BLK, LW), lambda i: (i, 0)),
        pl.BlockSpec((BLK, NF), lambda i: (i, 0)),
        pl.BlockSpec((BLK, NF), lambda i: (i, 0)),
        pl.BlockSpec((NF, LW), lambda i: (0, 0)),
        pl.BlockSpec((LW, NF * D), lambda i: (0, 0)),
        pl.BlockSpec((1,), lambda i: (0,)),
        pl.BlockSpec((NF * D, HID0), lambda i: (0, 0)),
        pl.BlockSpec((HID0,), lambda i: (0,)),
        pl.BlockSpec((HID0, HID1), lambda i: (0, 0)),
        pl.BlockSpec((HID1,), lambda i: (0,)),
        pl.BlockSpec((HID1, HID2), lambda i: (0, 0)),
        pl.BlockSpec((HID2,), lambda i: (0,)),
        pl.BlockSpec((1, HID2), lambda i: (0, 0)),
        pl.BlockSpec((1,), lambda i: (0,)),
    ],
    out_specs=pl.BlockSpec((BLK,), lambda i: (i,)),
    out_shape=jax.ShapeDtypeStruct((B,), jnp.float32),
)


def kernel(X, emb_table, dense_table, fm_bias, W0, b0, W1, b1, W2, b2, W3, b3):
    X = X.astype(jnp.int32)
    eq = (jnp.arange(NF, dtype=jnp.int32) * QPF)[None, :] + (X // 4)
    xa = X % 4
    idx = X + (jnp.arange(NF, dtype=jnp.int32) * V)[None, :]
    dq = idx // CHUNK
    xm = idx % CHUNK
    embT = emb_table.transpose(0, 2, 1).reshape(NF * D, V)
    ep = _repack(embT)
    dense_lin = jnp.pad(dense_table.reshape(NF * V), (0, NDR * CHUNK - NF * V))
    dense2d = dense_lin.reshape(NDR, CHUNK)
    rows, dvals = _build_sc_gather()(
        ep, dense2d, eq.reshape(NW, NCH, CHUNK), dq.reshape(NW, NCH, CHUNK))
    hq = rows.reshape(B, LW)
    dv = dvals.reshape(B, LW)
    fj = jnp.arange(LW, dtype=jnp.int32)
    exf = (fj[None, :] // CHUNK == jnp.arange(NF, dtype=jnp.int32)[:, None]
           ).astype(jnp.float32)
    mk = jnp.arange(NF * D, dtype=jnp.int32)
    m2 = ((fj[:, None] // CHUNK == mk[None, :] // D)
          & (fj[:, None] % D == mk[None, :] % D)).astype(jnp.bfloat16)
    return _mlp(hq, dv, xa, xm, exf, m2, fm_bias,
                W0.astype(jnp.bfloat16), b0, W1.astype(jnp.bfloat16), b1,
                W2.astype(jnp.bfloat16), b2, W3.reshape(1, HID2), b3)


# speedup vs baseline: 9.5074x; 6.1812x over previous
import functools

import jax
import jax.numpy as jnp
from jax import lax
from jax.experimental import pallas as pl
from jax.experimental.pallas import tpu as pltpu
from jax.experimental.pallas import tpu_sc as plsc

B = 4096
NF = 26
V = 100001
D = 32
BNF = B * NF
NC, NS = 2, 16
NW = NC * NS
BPW = BNF // NW
CHUNK = 128
NCH = BPW // CHUNK

QPF = 25088
XCH = 896
NXC = QPF // XCH
EPR = NF * QPF
NDR = (NF * V + CHUNK - 1) // CHUNK + 1

BLK = 512
HID0, HID1, HID2 = 1024, 512, 256
LW = NF * CHUNK


def _repack_body(i0_ref, i1_ref, i2_ref, i3_ref, o_ref):
    ei = lax.broadcasted_iota(jnp.int32, (D, D), 0)
    ej = lax.broadcasted_iota(jnp.int32, (D, D), 1)
    eye = (ei == ej).astype(jnp.float32)
    for a, r in enumerate((i0_ref, i1_ref, i2_ref, i3_ref)):
        t = lax.dot_general(r[...], eye, (((0,), (0,)), ((), ())),
                            preferred_element_type=jnp.float32)
        o_ref[:, D * a:D * (a + 1)] = t


_repack = pl.pallas_call(
    _repack_body,
    grid=(NF, NXC),
    in_specs=[
        pl.BlockSpec((D, XCH), lambda f, c: (f, 0 * NXC + c)),
        pl.BlockSpec((D, XCH), lambda f, c: (f, 1 * NXC + c)),
        pl.BlockSpec((D, XCH), lambda f, c: (f, 2 * NXC + c)),
        pl.BlockSpec((D, XCH), lambda f, c: (f, 3 * NXC + c)),
    ],
    out_specs=pl.BlockSpec((XCH, CHUNK), lambda f, c: (f * NXC + c, 0)),
    out_shape=jax.ShapeDtypeStruct((EPR, CHUNK), jnp.float32),
)


@functools.cache
def _build_sc_gather():
    mesh = plsc.VectorSubcoreMesh(core_axis_name="c", subcore_axis_name="s")

    @functools.partial(
        pl.kernel,
        out_type=(
            jax.ShapeDtypeStruct((BNF, CHUNK), jnp.float32),
            jax.ShapeDtypeStruct((BNF, CHUNK), jnp.float32),
        ),
        mesh=mesh,
        scratch_types=[
            pltpu.VMEM((NCH, CHUNK), jnp.int32),
            pltpu.VMEM((NCH, CHUNK), jnp.int32),
            pltpu.VMEM((CHUNK, CHUNK), jnp.float32),
            pltpu.VMEM((CHUNK, CHUNK), jnp.float32),
            pltpu.SemaphoreType.DMA,
            pltpu.SemaphoreType.DMA,
        ],
    )
    def _sc_gather(emb_hbm, dense_hbm, idx_hbm, idxd_hbm, oe_hbm, od_hbm,
                   idx_v, idxd_v, qb_v, db_v, sem_e, sem_d):
        wid = lax.axis_index("s") * NC + lax.axis_index("c")
        base = wid * BPW
        pltpu.sync_copy(idx_hbm.at[wid], idx_v)
        pltpu.sync_copy(idxd_hbm.at[wid], idxd_v)

        @pl.loop(0, NCH)
        def _(j):
            off = base + j * CHUNK
            cpe = pltpu.async_copy(emb_hbm.at[idx_v.at[j]], qb_v, sem_e)
            cpd = pltpu.async_copy(dense_hbm.at[idxd_v.at[j]], db_v, sem_d)
            cpe.wait()
            cpd.wait()
            pltpu.sync_copy(qb_v, oe_hbm.at[pl.ds(off, CHUNK)])
            pltpu.sync_copy(db_v, od_hbm.at[pl.ds(off, CHUNK)])

    return _sc_gather


def _mlp_body(hq_ref, dq_ref, xa_ref, xm_ref, exf_ref, m2_ref, fmb_ref,
              w0_ref, b0_ref, w1_ref, b1_ref, w2_ref, b2_ref, w3_ref, b3_ref,
              out_ref):
    exf = exf_ref[...]
    lane = lax.broadcasted_iota(jnp.int32, (BLK, LW), 1) % CHUNK
    a_exp = jnp.dot(xa_ref[...].astype(jnp.float32), exf,
                    preferred_element_type=jnp.float32)
    emask = (lane // D == a_exp.astype(jnp.int32)).astype(jnp.float32)
    hsel = (hq_ref[...] * emask).astype(jnp.bfloat16)
    h0 = jnp.dot(hsel, m2_ref[...], preferred_element_type=jnp.float32)
    m_exp = jnp.dot(xm_ref[...].astype(jnp.float32), exf,
                    preferred_element_type=jnp.float32)
    dmask = (lane == m_exp.astype(jnp.int32)).astype(jnp.float32)
    one = jnp.sum(dq_ref[...] * dmask, axis=1)
    ssq = jnp.sum(h0 * h0, axis=1)
    ki = lax.broadcasted_iota(jnp.int32, (NF * D, D), 0)
    di = lax.broadcasted_iota(jnp.int32, (NF * D, D), 1)
    fold = (ki % D == di).astype(jnp.bfloat16)
    h0b = h0.astype(jnp.bfloat16)
    s = jnp.dot(h0b, fold, preferred_element_type=jnp.float32)
    two = 0.5 * (jnp.sum(s * s, axis=1) - ssq)
    fm = one + two + fmb_ref[...]
    h = h0b
    for w_ref, b_ref in ((w0_ref, b0_ref), (w1_ref, b1_ref), (w2_ref, b2_ref)):
        hf = jnp.dot(h, w_ref[...], preferred_element_type=jnp.float32)
        h = jnp.maximum(hf + b_ref[...][None, :], 0.0).astype(jnp.bfloat16)
    hf = h.astype(jnp.float32)
    deep = jnp.maximum(jnp.sum(hf * w3_ref[...], axis=1) + b3_ref[...], 0.0)
    out_ref[...] = jax.nn.sigmoid(fm + deep)


_mlp = pl.pallas_call(
    _mlp_body,
    grid=(B // BLK,),
    in_specs=[
        pl.BlockSpec((BLK, LW), lambda i: (i, 0)),
        pl.BlockSpec((BLK, LW), lambda i: (i, 0)),
        pl.BlockSpec((BLK, NF), lambda i: (i, 0)),
        pl.BlockSpec((BLK, NF), lambda i: (i, 0)),
        pl.BlockSpec((NF, LW), lambda i: (0, 0)),
        pl.BlockSpec((LW, NF * D), lambda i: (0, 0)),
        pl.BlockSpec((1,), lambda i: (0,)),
        pl.BlockSpec((NF * D, HID0), lambda i: (0, 0)),
        pl.BlockSpec((HID0,), lambda i: (0,)),
        pl.BlockSpec((HID0, HID1), lambda i: (0, 0)),
        pl.BlockSpec((HID1,), lambda i: (0,)),
        pl.BlockSpec((HID1, HID2), lambda i: (0, 0)),
        pl.BlockSpec((HID2,), lambda i: (0,)),
        pl.BlockSpec((1, HID2), lambda i: (0, 0)),
        pl.BlockSpec((1,), lambda i: (0,)),
    ],
    out_specs=pl.BlockSpec((BLK,), lambda i: (i,)),
    out_shape=jax.ShapeDtypeStruct((B,), jnp.float32),
)


def kernel(X, emb_table, dense_table, fm_bias, W0, b0, W1, b1, W2, b2, W3, b3):
    X = X.astype(jnp.int32)
    eq = (jnp.arange(NF, dtype=jnp.int32) * QPF)[None, :] + (X % QPF)
    xa = X // QPF
    idx = X + (jnp.arange(NF, dtype=jnp.int32) * V)[None, :]
    dq = idx // CHUNK
    xm = idx % CHUNK
    embT = emb_table.transpose(0, 2, 1).reshape(NF * D, V)
    ep = _repack(embT, embT, embT, embT)
    dense_lin = jnp.pad(dense_table.reshape(NF * V), (0, NDR * CHUNK - NF * V))
    dense2d = dense_lin.reshape(NDR, CHUNK)
    rows, dvals = _build_sc_gather()(
        ep, dense2d, eq.reshape(NW, NCH, CHUNK), dq.reshape(NW, NCH, CHUNK))
    hq = rows.reshape(B, LW)
    dv = dvals.reshape(B, LW)
    fj = jnp.arange(LW, dtype=jnp.int32)
    exf = (fj[None, :] // CHUNK == jnp.arange(NF, dtype=jnp.int32)[:, None]
           ).astype(jnp.float32)
    mk = jnp.arange(NF * D, dtype=jnp.int32)
    m2 = ((fj[:, None] // CHUNK == mk[None, :] // D)
          & (fj[:, None] % D == mk[None, :] % D)).astype(jnp.bfloat16)
    return _mlp(hq, dv, xa, xm, exf, m2, fm_bias,
                W0.astype(jnp.bfloat16), b0, W1.astype(jnp.bfloat16), b1,
                W2.astype(jnp.bfloat16), b2, W3.reshape(1, HID2), b3)


# speedup vs baseline: 16.7666x; 1.7635x over previous
import functools

import jax
import jax.numpy as jnp
from jax import lax
from jax.experimental import pallas as pl
from jax.experimental.pallas import tpu as pltpu
from jax.experimental.pallas import tpu_sc as plsc

B = 4096
NF = 26
V = 100001
D = 32
BNF = B * NF
NC, NS = 2, 16
NW = NC * NS
BPW = BNF // NW
CHUNK = 128
NCH = BPW // CHUNK

QPF = 25088
XCH = 3584
NXC = QPF // XCH
EPR = NF * QPF
NDR = (NF * V + CHUNK - 1) // CHUNK + 1

BLK = 512
HID0, HID1, HID2 = 1024, 512, 256
LW = NF * CHUNK


def _repack_body(i0_ref, i1_ref, i2_ref, i3_ref, o_ref):
    ei = lax.broadcasted_iota(jnp.int32, (CHUNK, CHUNK), 0)
    ej = lax.broadcasted_iota(jnp.int32, (CHUNK, CHUNK), 1)
    eye = (ei == ej).astype(jnp.bfloat16)
    ins = jnp.concatenate(
        [r[...].astype(jnp.bfloat16) for r in (i0_ref, i1_ref, i2_ref, i3_ref)],
        axis=0)
    o_ref[...] = lax.dot_general(ins, eye, (((0,), (0,)), ((), ())),
                                 preferred_element_type=jnp.float32)


_repack = pl.pallas_call(
    _repack_body,
    grid=(NF, NXC),
    in_specs=[
        pl.BlockSpec((D, XCH), lambda f, c: (f, 0 * NXC + c)),
        pl.BlockSpec((D, XCH), lambda f, c: (f, 1 * NXC + c)),
        pl.BlockSpec((D, XCH), lambda f, c: (f, 2 * NXC + c)),
        pl.BlockSpec((D, XCH), lambda f, c: (f, 3 * NXC + c)),
    ],
    out_specs=pl.BlockSpec((XCH, CHUNK), lambda f, c: (f * NXC + c, 0)),
    out_shape=jax.ShapeDtypeStruct((EPR, CHUNK), jnp.float32),
)


@functools.cache
def _build_sc_gather():
    mesh = plsc.VectorSubcoreMesh(core_axis_name="c", subcore_axis_name="s")

    @functools.partial(
        pl.kernel,
        out_type=(
            jax.ShapeDtypeStruct((BNF, CHUNK), jnp.float32),
            jax.ShapeDtypeStruct((BNF, CHUNK), jnp.float32),
        ),
        mesh=mesh,
        scratch_types=[
            pltpu.VMEM((NCH, CHUNK), jnp.int32),
            pltpu.VMEM((NCH, CHUNK), jnp.int32),
            pltpu.VMEM((CHUNK, CHUNK), jnp.float32),
            pltpu.VMEM((CHUNK, CHUNK), jnp.float32),
            pltpu.SemaphoreType.DMA,
            pltpu.SemaphoreType.DMA,
        ],
    )
    def _sc_gather(emb_hbm, dense_hbm, idx_hbm, idxd_hbm, oe_hbm, od_hbm,
                   idx_v, idxd_v, qb_v, db_v, sem_e, sem_d):
        wid = lax.axis_index("s") * NC + lax.axis_index("c")
        base = wid * BPW
        pltpu.sync_copy(idx_hbm.at[wid], idx_v)
        pltpu.sync_copy(idxd_hbm.at[wid], idxd_v)

        @pl.loop(0, NCH)
        def _(j):
            off = base + j * CHUNK
            cpe = pltpu.async_copy(emb_hbm.at[idx_v.at[j]], qb_v, sem_e)
            cpd = pltpu.async_copy(dense_hbm.at[idxd_v.at[j]], db_v, sem_d)
            cpe.wait()
            cpd.wait()
            pltpu.sync_copy(qb_v, oe_hbm.at[pl.ds(off, CHUNK)])
            pltpu.sync_copy(db_v, od_hbm.at[pl.ds(off, CHUNK)])

    return _sc_gather


def _mlp_body(hq_ref, dq_ref, xa_ref, xm_ref, exf_ref, m2_ref, fmb_ref,
              w0_ref, b0_ref, w1_ref, b1_ref, w2_ref, b2_ref, w3_ref, b3_ref,
              out_ref):
    exf = exf_ref[...]
    lane = lax.broadcasted_iota(jnp.int32, (BLK, LW), 1) % CHUNK
    a_exp = jnp.dot(xa_ref[...].astype(jnp.float32), exf,
                    preferred_element_type=jnp.float32)
    emask = (lane // D == a_exp.astype(jnp.int32)).astype(jnp.float32)
    hsel = (hq_ref[...] * emask).astype(jnp.bfloat16)
    h0 = jnp.dot(hsel, m2_ref[...], preferred_element_type=jnp.float32)
    m_exp = jnp.dot(xm_ref[...].astype(jnp.float32), exf,
                    preferred_element_type=jnp.float32)
    dmask = (lane == m_exp.astype(jnp.int32)).astype(jnp.float32)
    one = jnp.sum(dq_ref[...] * dmask, axis=1)
    ssq = jnp.sum(h0 * h0, axis=1)
    ki = lax.broadcasted_iota(jnp.int32, (NF * D, D), 0)
    di = lax.broadcasted_iota(jnp.int32, (NF * D, D), 1)
    fold = (ki % D == di).astype(jnp.bfloat16)
    h0b = h0.astype(jnp.bfloat16)
    s = jnp.dot(h0b, fold, preferred_element_type=jnp.float32)
    two = 0.5 * (jnp.sum(s * s, axis=1) - ssq)
    fm = one + two + fmb_ref[...]
    h = h0b
    for w_ref, b_ref in ((w0_ref, b0_ref), (w1_ref, b1_ref), (w2_ref, b2_ref)):
        hf = jnp.dot(h, w_ref[...], preferred_element_type=jnp.float32)
        h = jnp.maximum(hf + b_ref[...][None, :], 0.0).astype(jnp.bfloat16)
    hf = h.astype(jnp.float32)
    deep = jnp.maximum(jnp.sum(hf * w3_ref[...], axis=1) + b3_ref[...], 0.0)
    out_ref[...] = jax.nn.sigmoid(fm + deep)


_mlp = pl.pallas_call(
    _mlp_body,
    grid=(B // BLK,),
    in_specs=[
        pl.BlockSpec((BLK, LW), lambda i: (i, 0)),
        pl.BlockSpec((BLK, LW), lambda i: (i, 0)),
        pl.BlockSpec((BLK, NF), lambda i: (i, 0)),
        pl.BlockSpec((BLK, NF), lambda i: (i, 0)),
        pl.BlockSpec((NF, LW), lambda i: (0, 0)),
        pl.BlockSpec((LW, NF * D), lambda i: (0, 0)),
        pl.BlockSpec((1,), lambda i: (0,)),
        pl.BlockSpec((NF * D, HID0), lambda i: (0, 0)),
        pl.BlockSpec((HID0,), lambda i: (0,)),
        pl.BlockSpec((HID0, HID1), lambda i: (0, 0)),
        pl.BlockSpec((HID1,), lambda i: (0,)),
        pl.BlockSpec((HID1, HID2), lambda i: (0, 0)),
        pl.BlockSpec((HID2,), lambda i: (0,)),
        pl.BlockSpec((1, HID2), lambda i: (0, 0)),
        pl.BlockSpec((1,), lambda i: (0,)),
    ],
    out_specs=pl.BlockSpec((BLK,), lambda i: (i,)),
    out_shape=jax.ShapeDtypeStruct((B,), jnp.float32),
)


def kernel(X, emb_table, dense_table, fm_bias, W0, b0, W1, b1, W2, b2, W3, b3):
    X = X.astype(jnp.int32)
    eq = (jnp.arange(NF, dtype=jnp.int32) * QPF)[None, :] + (X % QPF)
    xa = X // QPF
    idx = X + (jnp.arange(NF, dtype=jnp.int32) * V)[None, :]
    dq = idx // CHUNK
    xm = idx % CHUNK
    embT = emb_table.transpose(0, 2, 1).reshape(NF * D, V)
    ep = _repack(embT, embT, embT, embT)
    dense_lin = jnp.pad(dense_table.reshape(NF * V), (0, NDR * CHUNK - NF * V))
    dense2d = dense_lin.reshape(NDR, CHUNK)
    rows, dvals = _build_sc_gather()(
        ep, dense2d, eq.reshape(NW, NCH, CHUNK), dq.reshape(NW, NCH, CHUNK))
    hq = rows.reshape(B, LW)
    dv = dvals.reshape(B, LW)
    fj = jnp.arange(LW, dtype=jnp.int32)
    exf = (fj[None, :] // CHUNK == jnp.arange(NF, dtype=jnp.int32)[:, None]
           ).astype(jnp.float32)
    mk = jnp.arange(NF * D, dtype=jnp.int32)
    m2 = ((fj[:, None] // CHUNK == mk[None, :] // D)
          & (fj[:, None] % D == mk[None, :] % D)).astype(jnp.bfloat16)
    return _mlp(hq, dv, xa, xm, exf, m2, fm_bias,
                W0.astype(jnp.bfloat16), b0, W1.astype(jnp.bfloat16), b1,
                W2.astype(jnp.bfloat16), b2, W3.reshape(1, HID2), b3)


# speedup vs baseline: 18.1281x; 1.0812x over previous
import functools

import jax
import jax.numpy as jnp
from jax import lax
from jax.experimental import pallas as pl
from jax.experimental.pallas import tpu as pltpu
from jax.experimental.pallas import tpu_sc as plsc

B = 4096
NF = 26
V = 100001
D = 32
BNF = B * NF
NC, NS = 2, 16
NW = NC * NS
BPW = BNF // NW
CHUNK = 128
NCH = BPW // CHUNK

QPF = 25088
XCH = 12544
NXC = QPF // XCH
EPR = NF * QPF
NDR = (NF * V + CHUNK - 1) // CHUNK + 1

BLK = 512
HID0, HID1, HID2 = 1024, 512, 256
LW = NF * CHUNK


def _repack_body(i0_ref, i1_ref, i2_ref, i3_ref, o_ref):
    ei = lax.broadcasted_iota(jnp.int32, (CHUNK, CHUNK), 0)
    ej = lax.broadcasted_iota(jnp.int32, (CHUNK, CHUNK), 1)
    eye = (ei == ej).astype(jnp.bfloat16)
    ins = jnp.concatenate(
        [r[...].astype(jnp.bfloat16) for r in (i0_ref, i1_ref, i2_ref, i3_ref)],
        axis=0)
    o_ref[...] = lax.dot_general(ins, eye, (((0,), (0,)), ((), ())),
                                 preferred_element_type=jnp.float32)


_repack = pl.pallas_call(
    _repack_body,
    grid=(NF, NXC),
    in_specs=[
        pl.BlockSpec((D, XCH), lambda f, c: (f, 0 * NXC + c)),
        pl.BlockSpec((D, XCH), lambda f, c: (f, 1 * NXC + c)),
        pl.BlockSpec((D, XCH), lambda f, c: (f, 2 * NXC + c)),
        pl.BlockSpec((D, XCH), lambda f, c: (f, 3 * NXC + c)),
    ],
    out_specs=pl.BlockSpec((XCH, CHUNK), lambda f, c: (f * NXC + c, 0)),
    out_shape=jax.ShapeDtypeStruct((EPR, CHUNK), jnp.float32),
)


@functools.cache
def _build_sc_gather():
    mesh = plsc.VectorSubcoreMesh(core_axis_name="c", subcore_axis_name="s")

    @functools.partial(
        pl.kernel,
        out_type=(
            jax.ShapeDtypeStruct((BNF, CHUNK), jnp.float32),
            jax.ShapeDtypeStruct((BNF, CHUNK), jnp.float32),
        ),
        mesh=mesh,
        scratch_types=[
            pltpu.VMEM((NCH, CHUNK), jnp.int32),
            pltpu.VMEM((NCH, CHUNK), jnp.int32),
            pltpu.VMEM((CHUNK, CHUNK), jnp.float32),
            pltpu.VMEM((CHUNK, CHUNK), jnp.float32),
            pltpu.SemaphoreType.DMA,
            pltpu.SemaphoreType.DMA,
        ],
    )
    def _sc_gather(emb_hbm, dense_hbm, idx_hbm, idxd_hbm, oe_hbm, od_hbm,
                   idx_v, idxd_v, qb_v, db_v, sem_e, sem_d):
        wid = lax.axis_index("s") * NC + lax.axis_index("c")
        base = wid * BPW
        pltpu.sync_copy(idx_hbm.at[wid], idx_v)
        pltpu.sync_copy(idxd_hbm.at[wid], idxd_v)

        @pl.loop(0, NCH)
        def _(j):
            off = base + j * CHUNK
            cpe = pltpu.async_copy(emb_hbm.at[idx_v.at[j]], qb_v, sem_e)
            cpd = pltpu.async_copy(dense_hbm.at[idxd_v.at[j]], db_v, sem_d)
            cpe.wait()
            cpd.wait()
            pltpu.sync_copy(qb_v, oe_hbm.at[pl.ds(off, CHUNK)])
            pltpu.sync_copy(db_v, od_hbm.at[pl.ds(off, CHUNK)])

    return _sc_gather


def _mlp_body(hq_ref, dq_ref, xa_ref, xm_ref, exf_ref, m2_ref, fmb_ref,
              w0_ref, b0_ref, w1_ref, b1_ref, w2_ref, b2_ref, w3_ref, b3_ref,
              out_ref):
    exf = exf_ref[...]
    lane = lax.broadcasted_iota(jnp.int32, (BLK, LW), 1) % CHUNK
    a_exp = jnp.dot(xa_ref[...].astype(jnp.float32), exf,
                    preferred_element_type=jnp.float32)
    emask = (lane // D == a_exp.astype(jnp.int32)).astype(jnp.float32)
    hsel = (hq_ref[...] * emask).astype(jnp.bfloat16)
    h0 = jnp.dot(hsel, m2_ref[...], preferred_element_type=jnp.float32)
    m_exp = jnp.dot(xm_ref[...].astype(jnp.float32), exf,
                    preferred_element_type=jnp.float32)
    dmask = (lane == m_exp.astype(jnp.int32)).astype(jnp.float32)
    one = jnp.sum(dq_ref[...] * dmask, axis=1)
    ssq = jnp.sum(h0 * h0, axis=1)
    ki = lax.broadcasted_iota(jnp.int32, (NF * D, D), 0)
    di = lax.broadcasted_iota(jnp.int32, (NF * D, D), 1)
    fold = (ki % D == di).astype(jnp.bfloat16)
    h0b = h0.astype(jnp.bfloat16)
    s = jnp.dot(h0b, fold, preferred_element_type=jnp.float32)
    two = 0.5 * (jnp.sum(s * s, axis=1) - ssq)
    fm = one + two + fmb_ref[...]
    h = h0b
    for w_ref, b_ref in ((w0_ref, b0_ref), (w1_ref, b1_ref), (w2_ref, b2_ref)):
        hf = jnp.dot(h, w_ref[...], preferred_element_type=jnp.float32)
        h = jnp.maximum(hf + b_ref[...][None, :], 0.0).astype(jnp.bfloat16)
    hf = h.astype(jnp.float32)
    deep = jnp.maximum(jnp.sum(hf * w3_ref[...], axis=1) + b3_ref[...], 0.0)
    out_ref[...] = jax.nn.sigmoid(fm + deep)


_mlp = pl.pallas_call(
    _mlp_body,
    grid=(B // BLK,),
    in_specs=[
        pl.BlockSpec((BLK, LW), lambda i: (i, 0)),
        pl.BlockSpec((BLK, LW), lambda i: (i, 0)),
        pl.BlockSpec((BLK, NF), lambda i: (i, 0)),
        pl.BlockSpec((BLK, NF), lambda i: (i, 0)),
        pl.BlockSpec((NF, LW), lambda i: (0, 0)),
        pl.BlockSpec((LW, NF * D), lambda i: (0, 0)),
        pl.BlockSpec((1,), lambda i: (0,)),
        pl.BlockSpec((NF * D, HID0), lambda i: (0, 0)),
        pl.BlockSpec((HID0,), lambda i: (0,)),
        pl.BlockSpec((HID0, HID1), lambda i: (0, 0)),
        pl.BlockSpec((HID1,), lambda i: (0,)),
        pl.BlockSpec((HID1, HID2), lambda i: (0, 0)),
        pl.BlockSpec((HID2,), lambda i: (0,)),
        pl.BlockSpec((1, HID2), lambda i: (0, 0)),
        pl.BlockSpec((1,), lambda i: (0,)),
    ],
    out_specs=pl.BlockSpec((BLK,), lambda i: (i,)),
    out_shape=jax.ShapeDtypeStruct((B,), jnp.float32),
)


def kernel(X, emb_table, dense_table, fm_bias, W0, b0, W1, b1, W2, b2, W3, b3):
    X = X.astype(jnp.int32)
    eq = (jnp.arange(NF, dtype=jnp.int32) * QPF)[None, :] + (X % QPF)
    xa = X // QPF
    idx = X + (jnp.arange(NF, dtype=jnp.int32) * V)[None, :]
    dq = idx // CHUNK
    xm = idx % CHUNK
    embT = emb_table.transpose(0, 2, 1).reshape(NF * D, V)
    ep = _repack(embT, embT, embT, embT)
    dense_lin = jnp.pad(dense_table.reshape(NF * V), (0, NDR * CHUNK - NF * V))
    dense2d = dense_lin.reshape(NDR, CHUNK)
    rows, dvals = _build_sc_gather()(
        ep, dense2d, eq.reshape(NW, NCH, CHUNK), dq.reshape(NW, NCH, CHUNK))
    hq = rows.reshape(B, LW)
    dv = dvals.reshape(B, LW)
    fj = jnp.arange(LW, dtype=jnp.int32)
    exf = (fj[None, :] // CHUNK == jnp.arange(NF, dtype=jnp.int32)[:, None]
           ).astype(jnp.float32)
    mk = jnp.arange(NF * D, dtype=jnp.int32)
    m2 = ((fj[:, None] // CHUNK == mk[None, :] // D)
          & (fj[:, None] % D == mk[None, :] % D)).astype(jnp.bfloat16)
    return _mlp(hq, dv, xa, xm, exf, m2, fm_bias,
                W0.astype(jnp.bfloat16), b0, W1.astype(jnp.bfloat16), b1,
                W2.astype(jnp.bfloat16), b2, W3.reshape(1, HID2), b3)
